# baseline (speedup 1.0000x reference)
import functools

import jax
import jax.numpy as jnp
from jax import lax
from jax.experimental import pallas as pl
from jax.experimental.pallas import tpu as pltpu
from jax.experimental.pallas import tpu_sc as plsc

N = 10000
E = 320000
D_IN = 128
D_HID = 64
NG = 16
NC = 2
NS = 16
NW = NC * NS
CH = 128
NCH = 80
EPAD = NW * NCH * CH
NP = N + 128
RPT = 624
TAIL = N - RPT * NS


def _sliced_copy(src_ref, dst_ref, s):
    pltpu.sync_copy(src_ref.at[pl.ds(s * RPT, RPT)],
                    dst_ref.at[pl.ds(s * RPT, RPT)])

    @pl.when(s == NS - 1)
    def _():
        pltpu.sync_copy(src_ref.at[pl.ds(RPT * NS, TAIL)],
                        dst_ref.at[pl.ds(RPT * NS, TAIL)])

@functools.cache
def _mesh():
    return plsc.VectorSubcoreMesh(core_axis_name="c", subcore_axis_name="s",
                                  num_cores=NC, num_subcores=NS)


def _deg_body(dst_hbm, ones_hbm, zeros_hbm, out_hbm, idx_v, ones_v, acc_sh,
              sem):
    c = lax.axis_index("c")
    s = lax.axis_index("s")
    wid = c * NS + s
    _sliced_copy(zeros_hbm, acc_sh, s)
    pltpu.sync_copy(ones_hbm, ones_v)
    pltpu.sync_copy(dst_hbm.at[wid], idx_v)
    plsc.subcore_barrier()

    @pl.loop(0, NCH)
    def _(j):
        pltpu.sync_copy(ones_v, acc_sh.at[idx_v.at[j]], add=True)

    plsc.subcore_barrier()
    _sliced_copy(acc_sh, out_hbm.at[c], s)


@functools.cache
def _deg_call():
    return pl.kernel(
        _deg_body,
        out_type=jax.ShapeDtypeStruct((NC, N, 16), jnp.float32),
        mesh=_mesh(),
        scratch_types=[
            pltpu.VMEM((NCH, CH), jnp.int32),
            pltpu.VMEM((CH, 16), jnp.float32),
            pltpu.VMEM_SHARED((NP, 16), jnp.float32),
            pltpu.SemaphoreType.DMA,
        ],
        compiler_params=pltpu.CompilerParams(use_tc_tiling_on_sc=False),
    )


def _agg_body(h_hbm, src_hbm, dst_hbm, zeros_hbm, out_hbm, sidx_v, didx_v,
              rows0_v, rows1_v, rows2_v, rows3_v, acc_sh,
              sem0, sem1, sem2, sem3, sem4, sem5, sem6, sem7):
    c = lax.axis_index("c")
    s = lax.axis_index("s")
    wid = c * NS + s

    @pl.when(c == 0)
    def _():
        _sliced_copy(h_hbm, acc_sh, s)

    @pl.when(c != 0)
    def _():
        _sliced_copy(zeros_hbm, acc_sh, s)

    pltpu.sync_copy(src_hbm.at[wid], sidx_v)
    pltpu.sync_copy(dst_hbm.at[wid], didx_v)
    plsc.subcore_barrier()

    bufs = (rows0_v, rows1_v, rows2_v, rows3_v)
    gsems = (sem0, sem1, sem2, sem3)
    ssems = (sem4, sem5, sem6, sem7)

    def issue_g(k, b):
        pltpu.async_copy(h_hbm.at[sidx_v.at[k]], bufs[b], gsems[b])

    def wait_g(k, b):
        pltpu.make_async_copy(h_hbm.at[sidx_v.at[k]], bufs[b],
                              gsems[b]).wait()

    def issue_s(k, b):
        pltpu.async_copy(bufs[b], acc_sh.at[didx_v.at[k]], ssems[b],
                         add=True)

    def wait_s(k, b):
        pltpu.make_async_copy(bufs[b], acc_sh.at[didx_v.at[k]],
                              ssems[b]).wait()

    for b in range(4):
        issue_g(b, b)

    @pl.loop(0, NCH - 4, step=4)
    def _(j):
        for b in range(4):
            wait_g(j + b, b)
            issue_s(j + b, b)
        for b in range(4):
            wait_s(j + b, b)
            issue_g(j + b + 4, b)

    for b in range(4):
        wait_g(NCH - 4 + b, b)
        issue_s(NCH - 4 + b, b)
    for b in range(4):
        wait_s(NCH - 4 + b, b)

    plsc.subcore_barrier()
    _sliced_copy(acc_sh, out_hbm.at[c], s)


@functools.cache
def _agg_call():
    return pl.kernel(
        _agg_body,
        out_type=jax.ShapeDtypeStruct((NC, N, D_HID), jnp.float32),
        mesh=_mesh(),
        scratch_types=[
            pltpu.VMEM((NCH, CH), jnp.int32),
            pltpu.VMEM((NCH, CH), jnp.int32),
            pltpu.VMEM((CH, D_HID), jnp.float32),
            pltpu.VMEM((CH, D_HID), jnp.float32),
            pltpu.VMEM((CH, D_HID), jnp.float32),
            pltpu.VMEM((CH, D_HID), jnp.float32),
            pltpu.VMEM_SHARED((NP, D_HID), jnp.float32),
            pltpu.SemaphoreType.DMA,
            pltpu.SemaphoreType.DMA,
            pltpu.SemaphoreType.DMA,
            pltpu.SemaphoreType.DMA,
            pltpu.SemaphoreType.DMA,
            pltpu.SemaphoreType.DMA,
            pltpu.SemaphoreType.DMA,
            pltpu.SemaphoreType.DMA,
        ],
        compiler_params=pltpu.CompilerParams(use_tc_tiling_on_sc=False),
    )


def _dinv(d0_ref, d1_ref):
    deg = d0_ref[:, 0:1] + d1_ref[:, 0:1] + 1.0
    return lax.rsqrt(jnp.maximum(deg, 1.0))


def _p1_body(x_ref, w_ref, d0_ref, d1_ref, o_ref):
    dinv = _dinv(d0_ref, d1_ref)
    o_ref[...] = jnp.dot(x_ref[...], w_ref[...],
                         preferred_element_type=jnp.float32) * dinv


_p1_call = pl.pallas_call(
    _p1_body,
    out_shape=jax.ShapeDtypeStruct((N, D_HID), jnp.float32),
)


def _fuse_body(p_ref, w_ref, b_ref, d0_ref, d1_ref, o_ref):
    dinv = _dinv(d0_ref, d1_ref)
    h = jnp.maximum(dinv * (p_ref[0] + p_ref[1]) + b_ref[...], 0.0)
    o_ref[...] = jnp.dot(h, w_ref[...],
                         preferred_element_type=jnp.float32) * dinv


_fuse_call = pl.pallas_call(
    _fuse_body,
    out_shape=jax.ShapeDtypeStruct((N, D_HID), jnp.float32),
)


def _final_body(p_ref, b3_ref, d0_ref, d1_ref, batch_ref, wl1_ref, bl1_ref,
                wl2_ref, bl2_ref, o_ref):
    dinv = _dinv(d0_ref, d1_ref)
    h = jnp.maximum(dinv * (p_ref[0] + p_ref[1]) + b3_ref[...], 0.0)
    ids = batch_ref[...]
    gids = lax.broadcasted_iota(jnp.int32, (1, NG), 1)
    oh = (ids == gids).astype(jnp.float32)
    sums = lax.dot_general(oh, h, (((0,), (0,)), ((), ())),
                           preferred_element_type=jnp.float32)
    counts = jnp.sum(oh, axis=0)[:, None]
    pooled = sums / jnp.maximum(counts, 1.0)
    z = jnp.maximum(
        jnp.dot(pooled, wl1_ref[...], preferred_element_type=jnp.float32)
        + bl1_ref[...], 0.0)
    o_ref[...] = (jnp.dot(z, wl2_ref[...], preferred_element_type=jnp.float32)
                  + bl2_ref[...])


_final_call = pl.pallas_call(
    _final_body,
    out_shape=jax.ShapeDtypeStruct((NG, 1), jnp.float32),
)


@jax.jit
def kernel(x, edge_index, batch, W1, b1, W2, b2, W3, b3, Wl1, bl1, Wl2, bl2):
    ppt = (EPAD - E) // NW
    psrc = jnp.broadcast_to(jnp.arange(ppt, dtype=jnp.int32)[None],
                            (NW, ppt))
    pdst = N + (psrc % 128)
    src = jnp.concatenate(
        [edge_index[0].reshape(NW, E // NW), psrc], axis=1
    ).reshape(NW, NCH, CH)
    dst = jnp.concatenate(
        [edge_index[1].reshape(NW, E // NW), pdst], axis=1
    ).reshape(NW, NCH, CH)
    zeros64 = jnp.zeros((N, D_HID), jnp.float32)
    zeros16 = jnp.zeros((N, 16), jnp.float32)
    ones16 = jnp.ones((CH, 16), jnp.float32)

    degp = _deg_call()(dst, ones16, zeros16)
    d0 = degp[0]
    d1 = degp[1]

    h1t = _p1_call(x, W1, d0, d1)
    p = _agg_call()(h1t, src, dst, zeros64)
    h2t = _fuse_call(p, W2, b1[None, :], d0, d1)
    p = _agg_call()(h2t, src, dst, zeros64)
    h3t = _fuse_call(p, W3, b2[None, :], d0, d1)
    p = _agg_call()(h3t, src, dst, zeros64)
    out = _final_call(p, b3[None, :], d0, d1, batch[:, None],
                      Wl1, bl1[None, :], Wl2, bl2[None, :])
    return out


# speedup vs baseline: 1.0828x; 1.0828x over previous
import functools

import jax
import jax.numpy as jnp
from jax import lax
from jax.experimental import pallas as pl
from jax.experimental.pallas import tpu as pltpu
from jax.experimental.pallas import tpu_sc as plsc

N = 10000
E = 320000
D_IN = 128
D_HID = 64
NG = 16
NC = 2
NS = 16
NW = NC * NS
CH = 160
NCH = 64
EPAD = NW * NCH * CH
NP = N + 128
RPT = 624
TAIL = N - RPT * NS


def _sliced_copy(src_ref, dst_ref, s):
    pltpu.sync_copy(src_ref.at[pl.ds(s * RPT, RPT)],
                    dst_ref.at[pl.ds(s * RPT, RPT)])

    @pl.when(s == NS - 1)
    def _():
        pltpu.sync_copy(src_ref.at[pl.ds(RPT * NS, TAIL)],
                        dst_ref.at[pl.ds(RPT * NS, TAIL)])

@functools.cache
def _mesh():
    return plsc.VectorSubcoreMesh(core_axis_name="c", subcore_axis_name="s",
                                  num_cores=NC, num_subcores=NS)


def _deg_body(dst_hbm, ones_hbm, zeros_hbm, out_hbm, idx_v, ones_v, acc_sh,
              sem):
    c = lax.axis_index("c")
    s = lax.axis_index("s")
    wid = c * NS + s
    _sliced_copy(zeros_hbm, acc_sh, s)
    pltpu.sync_copy(ones_hbm, ones_v)
    pltpu.sync_copy(dst_hbm.at[wid], idx_v)
    plsc.subcore_barrier()

    @pl.loop(0, NCH)
    def _(j):
        pltpu.sync_copy(ones_v, acc_sh.at[idx_v.at[j]], add=True)

    plsc.subcore_barrier()
    _sliced_copy(acc_sh, out_hbm.at[c], s)


@functools.cache
def _deg_call():
    return pl.kernel(
        _deg_body,
        out_type=jax.ShapeDtypeStruct((NC, N, 16), jnp.float32),
        mesh=_mesh(),
        scratch_types=[
            pltpu.VMEM((NCH, CH), jnp.int32),
            pltpu.VMEM((CH, 16), jnp.float32),
            pltpu.VMEM_SHARED((NP, 16), jnp.float32),
            pltpu.SemaphoreType.DMA,
        ],
        compiler_params=pltpu.CompilerParams(use_tc_tiling_on_sc=False),
    )


def _agg_body(h_hbm, src_hbm, dst_hbm, zeros_hbm, out_hbm, sidx_v, didx_v,
              rows0_v, rows1_v, rows2_v, rows3_v, acc_sh,
              sem0, sem1, sem2, sem3):
    c = lax.axis_index("c")
    s = lax.axis_index("s")
    wid = c * NS + s

    @pl.when(c == 0)
    def _():
        _sliced_copy(h_hbm, acc_sh, s)

    @pl.when(c != 0)
    def _():
        _sliced_copy(zeros_hbm, acc_sh, s)

    pltpu.sync_copy(src_hbm.at[wid], sidx_v)
    pltpu.sync_copy(dst_hbm.at[wid], didx_v)
    plsc.subcore_barrier()

    bufs = (rows0_v, rows1_v, rows2_v, rows3_v)
    sems = (sem0, sem1, sem2, sem3)

    def issue(k, b):
        pltpu.async_copy(h_hbm.at[sidx_v.at[k]], bufs[b], sems[b])

    def drain_scatter(k, b):
        pltpu.make_async_copy(h_hbm.at[sidx_v.at[k]], bufs[b],
                              sems[b]).wait()
        pltpu.sync_copy(bufs[b], acc_sh.at[didx_v.at[k]], add=True)

    issue(0, 0)
    issue(1, 1)
    issue(2, 2)

    @pl.loop(0, NCH - 4, step=4)
    def _(j):
        issue(j + 3, 3)
        drain_scatter(j, 0)
        issue(j + 4, 0)
        drain_scatter(j + 1, 1)
        issue(j + 5, 1)
        drain_scatter(j + 2, 2)
        issue(j + 6, 2)
        drain_scatter(j + 3, 3)

    issue(NCH - 1, 3)
    drain_scatter(NCH - 4, 0)
    drain_scatter(NCH - 3, 1)
    drain_scatter(NCH - 2, 2)
    drain_scatter(NCH - 1, 3)

    plsc.subcore_barrier()
    _sliced_copy(acc_sh, out_hbm.at[c], s)


@functools.cache
def _agg_call():
    return pl.kernel(
        _agg_body,
        out_type=jax.ShapeDtypeStruct((NC, N, D_HID), jnp.float32),
        mesh=_mesh(),
        scratch_types=[
            pltpu.VMEM((NCH, CH), jnp.int32),
            pltpu.VMEM((NCH, CH), jnp.int32),
            pltpu.VMEM((CH, D_HID), jnp.float32),
            pltpu.VMEM((CH, D_HID), jnp.float32),
            pltpu.VMEM((CH, D_HID), jnp.float32),
            pltpu.VMEM((CH, D_HID), jnp.float32),
            pltpu.VMEM_SHARED((NP, D_HID), jnp.float32),
            pltpu.SemaphoreType.DMA,
            pltpu.SemaphoreType.DMA,
            pltpu.SemaphoreType.DMA,
            pltpu.SemaphoreType.DMA,
        ],
        compiler_params=pltpu.CompilerParams(use_tc_tiling_on_sc=False),
    )


def _dinv(d0_ref, d1_ref):
    deg = d0_ref[:, 0:1] + d1_ref[:, 0:1] + 1.0
    return lax.rsqrt(jnp.maximum(deg, 1.0))


def _p1_body(x_ref, w_ref, d0_ref, d1_ref, o_ref):
    dinv = _dinv(d0_ref, d1_ref)
    o_ref[...] = jnp.dot(x_ref[...], w_ref[...],
                         preferred_element_type=jnp.float32) * dinv


_p1_call = pl.pallas_call(
    _p1_body,
    out_shape=jax.ShapeDtypeStruct((N, D_HID), jnp.float32),
)


def _fuse_body(p_ref, w_ref, b_ref, d0_ref, d1_ref, o_ref):
    dinv = _dinv(d0_ref, d1_ref)
    h = jnp.maximum(dinv * (p_ref[0] + p_ref[1]) + b_ref[...], 0.0)
    o_ref[...] = jnp.dot(h, w_ref[...],
                         preferred_element_type=jnp.float32) * dinv


_fuse_call = pl.pallas_call(
    _fuse_body,
    out_shape=jax.ShapeDtypeStruct((N, D_HID), jnp.float32),
)


def _final_body(p_ref, b3_ref, d0_ref, d1_ref, batch_ref, wl1_ref, bl1_ref,
                wl2_ref, bl2_ref, o_ref):
    dinv = _dinv(d0_ref, d1_ref)
    h = jnp.maximum(dinv * (p_ref[0] + p_ref[1]) + b3_ref[...], 0.0)
    ids = batch_ref[...]
    gids = lax.broadcasted_iota(jnp.int32, (1, NG), 1)
    oh = (ids == gids).astype(jnp.float32)
    sums = lax.dot_general(oh, h, (((0,), (0,)), ((), ())),
                           preferred_element_type=jnp.float32)
    counts = jnp.sum(oh, axis=0)[:, None]
    pooled = sums / jnp.maximum(counts, 1.0)
    z = jnp.maximum(
        jnp.dot(pooled, wl1_ref[...], preferred_element_type=jnp.float32)
        + bl1_ref[...], 0.0)
    o_ref[...] = (jnp.dot(z, wl2_ref[...], preferred_element_type=jnp.float32)
                  + bl2_ref[...])


_final_call = pl.pallas_call(
    _final_body,
    out_shape=jax.ShapeDtypeStruct((NG, 1), jnp.float32),
)


@jax.jit
def kernel(x, edge_index, batch, W1, b1, W2, b2, W3, b3, Wl1, bl1, Wl2, bl2):
    ppt = (EPAD - E) // NW
    psrc = jnp.broadcast_to(jnp.arange(ppt, dtype=jnp.int32)[None],
                            (NW, ppt))
    pdst = N + (psrc % 128)
    src = jnp.concatenate(
        [edge_index[0].reshape(NW, E // NW), psrc], axis=1
    ).reshape(NW, NCH, CH)
    dst = jnp.concatenate(
        [edge_index[1].reshape(NW, E // NW), pdst], axis=1
    ).reshape(NW, NCH, CH)
    zeros64 = jnp.zeros((N, D_HID), jnp.float32)
    zeros16 = jnp.zeros((N, 16), jnp.float32)
    ones16 = jnp.ones((CH, 16), jnp.float32)

    degp = _deg_call()(dst, ones16, zeros16)
    d0 = degp[0]
    d1 = degp[1]

    h1t = _p1_call(x, W1, d0, d1)
    p = _agg_call()(h1t, src, dst, zeros64)
    h2t = _fuse_call(p, W2, b1[None, :], d0, d1)
    p = _agg_call()(h2t, src, dst, zeros64)
    h3t = _fuse_call(p, W3, b2[None, :], d0, d1)
    p = _agg_call()(h3t, src, dst, zeros64)
    out = _final_call(p, b3[None, :], d0, d1, batch[:, None],
                      Wl1, bl1[None, :], Wl2, bl2[None, :])
    return out


# speedup vs baseline: 1.0857x; 1.0027x over previous
import functools

import jax
import jax.numpy as jnp
from jax import lax
from jax.experimental import pallas as pl
from jax.experimental.pallas import tpu as pltpu
from jax.experimental.pallas import tpu_sc as plsc

N = 10000
E = 320000
D_IN = 128
D_HID = 64
NG = 16
NC = 2
NS = 16
NW = NC * NS
CH = 256
NCH = 40
EPAD = NW * NCH * CH
NP = N + 128
RPT = 624
TAIL = N - RPT * NS


def _sliced_copy(src_ref, dst_ref, s):
    pltpu.sync_copy(src_ref.at[pl.ds(s * RPT, RPT)],
                    dst_ref.at[pl.ds(s * RPT, RPT)])

    @pl.when(s == NS - 1)
    def _():
        pltpu.sync_copy(src_ref.at[pl.ds(RPT * NS, TAIL)],
                        dst_ref.at[pl.ds(RPT * NS, TAIL)])

@functools.cache
def _mesh():
    return plsc.VectorSubcoreMesh(core_axis_name="c", subcore_axis_name="s",
                                  num_cores=NC, num_subcores=NS)


def _deg_body(dst_hbm, ones_hbm, zeros_hbm, out_hbm, idx_v, ones_v, acc_sh,
              sem):
    c = lax.axis_index("c")
    s = lax.axis_index("s")
    wid = c * NS + s
    _sliced_copy(zeros_hbm, acc_sh, s)
    pltpu.sync_copy(ones_hbm, ones_v)
    pltpu.sync_copy(dst_hbm.at[wid], idx_v)
    plsc.subcore_barrier()

    @pl.loop(0, NCH)
    def _(j):
        pltpu.sync_copy(ones_v, acc_sh.at[idx_v.at[j]], add=True)

    plsc.subcore_barrier()
    _sliced_copy(acc_sh, out_hbm.at[c], s)


@functools.cache
def _deg_call():
    return pl.kernel(
        _deg_body,
        out_type=jax.ShapeDtypeStruct((NC, N, 16), jnp.float32),
        mesh=_mesh(),
        scratch_types=[
            pltpu.VMEM((NCH, CH), jnp.int32),
            pltpu.VMEM((CH, 16), jnp.float32),
            pltpu.VMEM_SHARED((NP, 16), jnp.float32),
            pltpu.SemaphoreType.DMA,
        ],
        compiler_params=pltpu.CompilerParams(use_tc_tiling_on_sc=False),
    )


def _agg_body(h_hbm, src_hbm, dst_hbm, zeros_hbm, out_hbm, sidx_v, didx_v,
              rows0_v, rows1_v, rows2_v, rows3_v, acc_sh,
              sem0, sem1, sem2, sem3):
    c = lax.axis_index("c")
    s = lax.axis_index("s")
    wid = c * NS + s

    @pl.when(c == 0)
    def _():
        _sliced_copy(h_hbm, acc_sh, s)

    @pl.when(c != 0)
    def _():
        _sliced_copy(zeros_hbm, acc_sh, s)

    pltpu.sync_copy(src_hbm.at[wid], sidx_v)
    pltpu.sync_copy(dst_hbm.at[wid], didx_v)
    plsc.subcore_barrier()

    bufs = (rows0_v, rows1_v, rows2_v, rows3_v)
    sems = (sem0, sem1, sem2, sem3)

    def issue(k, b):
        pltpu.async_copy(h_hbm.at[sidx_v.at[k]], bufs[b], sems[b])

    def drain_scatter(k, b):
        pltpu.make_async_copy(h_hbm.at[sidx_v.at[k]], bufs[b],
                              sems[b]).wait()
        pltpu.sync_copy(bufs[b], acc_sh.at[didx_v.at[k]], add=True)

    issue(0, 0)
    issue(1, 1)
    issue(2, 2)

    @pl.loop(0, NCH - 4, step=4)
    def _(j):
        issue(j + 3, 3)
        drain_scatter(j, 0)
        issue(j + 4, 0)
        drain_scatter(j + 1, 1)
        issue(j + 5, 1)
        drain_scatter(j + 2, 2)
        issue(j + 6, 2)
        drain_scatter(j + 3, 3)

    issue(NCH - 1, 3)
    drain_scatter(NCH - 4, 0)
    drain_scatter(NCH - 3, 1)
    drain_scatter(NCH - 2, 2)
    drain_scatter(NCH - 1, 3)

    plsc.subcore_barrier()
    _sliced_copy(acc_sh, out_hbm.at[c], s)


@functools.cache
def _agg_call():
    return pl.kernel(
        _agg_body,
        out_type=jax.ShapeDtypeStruct((NC, N, D_HID), jnp.float32),
        mesh=_mesh(),
        scratch_types=[
            pltpu.VMEM((NCH, CH), jnp.int32),
            pltpu.VMEM((NCH, CH), jnp.int32),
            pltpu.VMEM((CH, D_HID), jnp.float32),
            pltpu.VMEM((CH, D_HID), jnp.float32),
            pltpu.VMEM((CH, D_HID), jnp.float32),
            pltpu.VMEM((CH, D_HID), jnp.float32),
            pltpu.VMEM_SHARED((NP, D_HID), jnp.float32),
            pltpu.SemaphoreType.DMA,
            pltpu.SemaphoreType.DMA,
            pltpu.SemaphoreType.DMA,
            pltpu.SemaphoreType.DMA,
        ],
        compiler_params=pltpu.CompilerParams(use_tc_tiling_on_sc=False),
    )


def _dinv(d0_ref, d1_ref):
    deg = d0_ref[:, 0:1] + d1_ref[:, 0:1] + 1.0
    return lax.rsqrt(jnp.maximum(deg, 1.0))


def _p1_body(x_ref, w_ref, d0_ref, d1_ref, o_ref):
    dinv = _dinv(d0_ref, d1_ref)
    o_ref[...] = jnp.dot(x_ref[...], w_ref[...],
                         preferred_element_type=jnp.float32) * dinv


_p1_call = pl.pallas_call(
    _p1_body,
    out_shape=jax.ShapeDtypeStruct((N, D_HID), jnp.float32),
)


def _fuse_body(p_ref, w_ref, b_ref, d0_ref, d1_ref, o_ref):
    dinv = _dinv(d0_ref, d1_ref)
    h = jnp.maximum(dinv * (p_ref[0] + p_ref[1]) + b_ref[...], 0.0)
    o_ref[...] = jnp.dot(h, w_ref[...],
                         preferred_element_type=jnp.float32) * dinv


_fuse_call = pl.pallas_call(
    _fuse_body,
    out_shape=jax.ShapeDtypeStruct((N, D_HID), jnp.float32),
)


def _final_body(p_ref, b3_ref, d0_ref, d1_ref, batch_ref, wl1_ref, bl1_ref,
                wl2_ref, bl2_ref, o_ref):
    dinv = _dinv(d0_ref, d1_ref)
    h = jnp.maximum(dinv * (p_ref[0] + p_ref[1]) + b3_ref[...], 0.0)
    ids = batch_ref[...]
    gids = lax.broadcasted_iota(jnp.int32, (1, NG), 1)
    oh = (ids == gids).astype(jnp.float32)
    sums = lax.dot_general(oh, h, (((0,), (0,)), ((), ())),
                           preferred_element_type=jnp.float32)
    counts = jnp.sum(oh, axis=0)[:, None]
    pooled = sums / jnp.maximum(counts, 1.0)
    z = jnp.maximum(
        jnp.dot(pooled, wl1_ref[...], preferred_element_type=jnp.float32)
        + bl1_ref[...], 0.0)
    o_ref[...] = (jnp.dot(z, wl2_ref[...], preferred_element_type=jnp.float32)
                  + bl2_ref[...])


_final_call = pl.pallas_call(
    _final_body,
    out_shape=jax.ShapeDtypeStruct((NG, 1), jnp.float32),
)


@jax.jit
def kernel(x, edge_index, batch, W1, b1, W2, b2, W3, b3, Wl1, bl1, Wl2, bl2):
    ppt = (EPAD - E) // NW
    psrc = jnp.broadcast_to(jnp.arange(ppt, dtype=jnp.int32)[None],
                            (NW, ppt))
    pdst = N + (psrc % 128)
    src = jnp.concatenate(
        [edge_index[0].reshape(NW, E // NW), psrc], axis=1
    ).reshape(NW, NCH, CH)
    dst = jnp.concatenate(
        [edge_index[1].reshape(NW, E // NW), pdst], axis=1
    ).reshape(NW, NCH, CH)
    zeros64 = jnp.zeros((N, D_HID), jnp.float32)
    zeros16 = jnp.zeros((N, 16), jnp.float32)
    ones16 = jnp.ones((CH, 16), jnp.float32)

    degp = _deg_call()(dst, ones16, zeros16)
    d0 = degp[0]
    d1 = degp[1]

    h1t = _p1_call(x, W1, d0, d1)
    p = _agg_call()(h1t, src, dst, zeros64)
    h2t = _fuse_call(p, W2, b1[None, :], d0, d1)
    p = _agg_call()(h2t, src, dst, zeros64)
    h3t = _fuse_call(p, W3, b2[None, :], d0, d1)
    p = _agg_call()(h3t, src, dst, zeros64)
    out = _final_call(p, b3[None, :], d0, d1, batch[:, None],
                      Wl1, bl1[None, :], Wl2, bl2[None, :])
    return out
